# baseline (speedup 1.0000x reference)
import functools

import numpy as np
import jax
import jax.numpy as jnp
from jax import lax
from jax.experimental import pallas as pl
from jax.experimental.pallas import tpu as pltpu
from jax.experimental.pallas import tpu_sc as plsc

MAXD = 64


def _mm_body(f_ref, w_ref, b_ref, o_ref):
    o_ref[...] = (
        jnp.dot(f_ref[...], w_ref[...], preferred_element_type=jnp.float32)
        + b_ref[...]
    )


def _matmul(feat, weight, bias):
    n, di = feat.shape
    do = weight.shape[1]
    rb = 1000
    return pl.pallas_call(
        _mm_body,
        grid=(n // rb,),
        in_specs=[
            pl.BlockSpec((rb, di), lambda i: (i, 0)),
            pl.BlockSpec((di, do), lambda i: (0, 0)),
            pl.BlockSpec((1, do), lambda i: (0, 0)),
        ],
        out_specs=pl.BlockSpec((rb, do), lambda i: (i, 0)),
        out_shape=jax.ShapeDtypeStruct((n, do), jnp.float32),
    )(feat, weight, bias.reshape(1, do))


def _sc_gather(h_ext, idx_flat):
    info = plsc.get_sparse_core_info()
    nw = info.num_cores * info.num_subcores
    b = idx_flat.shape[0]
    d = h_ext.shape[1]
    b_per_w = b // nw
    k = 16
    ch = 32
    sup = k * ch
    nsup = b_per_w // sup
    mesh = plsc.VectorSubcoreMesh(core_axis_name="c", subcore_axis_name="s")

    @functools.partial(
        pl.kernel,
        out_type=jax.ShapeDtypeStruct((b, d), jnp.float32),
        mesh=mesh,
        scratch_types=[
            pltpu.VMEM((b_per_w,), jnp.int32),
            pltpu.VMEM((sup, d), jnp.float32),
            pltpu.SemaphoreType.DMA,
        ],
    )
    def gk(h_hbm, idx_hbm, out_hbm, idx_v, buf, gsem):
        wid = lax.axis_index("s") * info.num_cores + lax.axis_index("c")
        base0 = wid * b_per_w
        pltpu.sync_copy(idx_hbm.at[pl.ds(base0, b_per_w)], idx_v)

        def body(t, carry):
            base = t * sup
            for kk in range(k):
                pltpu.async_copy(
                    h_hbm.at[idx_v.at[pl.ds(base + kk * ch, ch)]],
                    buf.at[pl.ds(kk * ch, ch)],
                    gsem,
                )
            for kk in range(k):
                pltpu.make_async_copy(
                    h_hbm.at[idx_v.at[pl.ds(base + kk * ch, ch)]],
                    buf.at[pl.ds(kk * ch, ch)],
                    gsem,
                ).wait()
            pltpu.sync_copy(buf, out_hbm.at[pl.ds(base0 + base, sup)])
            return carry

        lax.fori_loop(0, nsup, body, 0)

    return gk(h_ext, idx_flat)


def _bitonic64(v):
    n = v.shape[0]
    d = v.shape[1]
    iota = lax.broadcasted_iota(jnp.int32, (n, 1), 0)
    k = 2
    while k <= n:
        j = k // 2
        while j >= 1:
            vb = v.reshape(n // (2 * j), 2, j, d)
            partner = jnp.concatenate(
                (vb[:, 1:2], vb[:, 0:1]), axis=1
            ).reshape(n, d)
            take_min = ((iota & k) == 0) == ((iota & j) == 0)
            v = jnp.where(
                take_min,
                jnp.minimum(v, partner),
                jnp.maximum(v, partner),
            )
            j //= 2
        k *= 2
    return v


def _med_body(nb, k_ref, g_ref, out_ref):
    i = pl.program_id(0)
    d = g_ref.shape[1]
    for j in range(nb):
        v = g_ref[j * MAXD:(j + 1) * MAXD, :]
        vs = _bitonic64(v)
        kj = k_ref[i * nb + j]
        eq = lax.broadcasted_iota(jnp.int32, (MAXD, d), 0) == kj
        out_ref[j, :] = jnp.sum(jnp.where(eq, vs, 0.0), axis=0)


def _median(g, karr):
    n = karr.shape[0]
    d = g.shape[1]
    nb = 8
    grid_spec = pltpu.PrefetchScalarGridSpec(
        num_scalar_prefetch=1,
        grid=(n // nb,),
        in_specs=[pl.BlockSpec((nb * MAXD, d), lambda i, k: (i, 0))],
        out_specs=pl.BlockSpec((nb, d), lambda i, k: (i, 0)),
    )
    return pl.pallas_call(
        functools.partial(_med_body, nb),
        grid_spec=grid_spec,
        out_shape=jax.ShapeDtypeStruct((n, d), jnp.float32),
    )(karr, g)


def kernel(feat, edge_index, weight, bias):
    n, _ = feat.shape
    do = weight.shape[1]
    e = edge_index.shape[1]
    src = edge_index[0]
    dst = edge_index[1]
    loop = jnp.arange(n, dtype=src.dtype)
    src_a = jnp.concatenate([src, loop])
    dst_a = jnp.concatenate([dst, loop])
    ep = e + n

    _, srcs = lax.sort_key_val(dst_a, src_a)
    counts = jnp.zeros((n,), jnp.int32).at[dst_a].add(1)
    offsets = jnp.cumsum(counts) - counts
    ceff = jnp.minimum(counts, MAXD)
    karr = ((ceff - 1) // 2).astype(jnp.int32)

    lane = jnp.arange(MAXD, dtype=jnp.int32)[None, :]
    pos = offsets[:, None] + lane
    valid = lane < counts[:, None]
    idx = jnp.where(valid, srcs[jnp.clip(pos, 0, ep - 1)], n).astype(jnp.int32)
    idx_flat = idx.reshape(-1)
    bp = 32 * 512 * -(-idx_flat.shape[0] // (32 * 512))
    idx_flat = jnp.concatenate(
        [idx_flat, jnp.full((bp - idx_flat.shape[0],), n, jnp.int32)]
    )

    h = _matmul(feat, weight, bias)
    h_ext = jnp.concatenate([h, jnp.full((1, do), jnp.inf, jnp.float32)])
    g = _sc_gather(h_ext, idx_flat)
    return _median(g, karr)


# speedup vs baseline: 4.6038x; 4.6038x over previous
import functools

import numpy as np
import jax
import jax.numpy as jnp
from jax import lax
from jax.experimental import pallas as pl
from jax.experimental.pallas import tpu as pltpu
from jax.experimental.pallas import tpu_sc as plsc

MAXD = 64


def _mm_body(f_ref, w_ref, b_ref, o_ref):
    o_ref[...] = (
        jnp.dot(f_ref[...], w_ref[...], preferred_element_type=jnp.float32)
        + b_ref[...]
    )


def _matmul(feat, weight, bias):
    n, di = feat.shape
    do = weight.shape[1]
    rb = 1000
    return pl.pallas_call(
        _mm_body,
        grid=(n // rb,),
        in_specs=[
            pl.BlockSpec((rb, di), lambda i: (i, 0)),
            pl.BlockSpec((di, do), lambda i: (0, 0)),
            pl.BlockSpec((1, do), lambda i: (0, 0)),
        ],
        out_specs=pl.BlockSpec((rb, do), lambda i: (i, 0)),
        out_shape=jax.ShapeDtypeStruct((n, do), jnp.float32),
    )(feat, weight, bias.reshape(1, do))


def _sc_gather(h_ext, idx_flat):
    info = plsc.get_sparse_core_info()
    nw = info.num_cores * info.num_subcores
    b = idx_flat.shape[0]
    d = h_ext.shape[1]
    b_per_w = b // nw
    k = 12
    ch = 16
    sup = k * ch
    nsup = b_per_w // sup
    mesh = plsc.VectorSubcoreMesh(core_axis_name="c", subcore_axis_name="s")

    nh = h_ext.shape[0]
    per_tile = nh // info.num_subcores

    @functools.partial(
        pl.kernel,
        out_type=jax.ShapeDtypeStruct((b, d), jnp.float32),
        mesh=mesh,
        scratch_types=[
            pltpu.VMEM((b_per_w,), jnp.int32),
            pltpu.VMEM((sup, d), jnp.float32),
            pltpu.VMEM_SHARED((nh, d), jnp.float32),
            pltpu.SemaphoreType.DMA,
        ],
    )
    def gk(h_hbm, idx_hbm, out_hbm, idx_v, buf, hs, gsem):
        sid = lax.axis_index("s")
        wid = sid * info.num_cores + lax.axis_index("c")
        base0 = wid * b_per_w

        tbase = sid * per_tile
        off = 0
        while off < per_tile:
            step = min(sup, per_tile - off)
            pltpu.sync_copy(h_hbm.at[pl.ds(tbase + off, step)],
                            buf.at[pl.ds(0, step)])
            pltpu.sync_copy(buf.at[pl.ds(0, step)],
                            hs.at[pl.ds(tbase + off, step)])
            off += step
        pltpu.sync_copy(idx_hbm.at[pl.ds(base0, b_per_w)], idx_v)
        plsc.subcore_barrier()

        def body(t, carry):
            base = t * sup
            for kk in range(k):
                pltpu.async_copy(
                    hs.at[idx_v.at[pl.ds(base + kk * ch, ch)]],
                    buf.at[pl.ds(kk * ch, ch)],
                    gsem,
                )
            for kk in range(k):
                pltpu.make_async_copy(
                    hs.at[idx_v.at[pl.ds(base + kk * ch, ch)]],
                    buf.at[pl.ds(kk * ch, ch)],
                    gsem,
                ).wait()
            pltpu.sync_copy(buf, out_hbm.at[pl.ds(base0 + base, sup)])
            return carry

        lax.fori_loop(0, nsup, body, 0)

    return gk(h_ext, idx_flat)


def _bitonic64(v):
    n = v.shape[0]
    d = v.shape[1]
    iota = lax.broadcasted_iota(jnp.int32, (n, 1), 0)
    k = 2
    while k <= n:
        j = k // 2
        while j >= 1:
            vb = v.reshape(n // (2 * j), 2, j, d)
            partner = jnp.concatenate(
                (vb[:, 1:2], vb[:, 0:1]), axis=1
            ).reshape(n, d)
            take_min = ((iota & k) == 0) == ((iota & j) == 0)
            v = jnp.where(
                take_min,
                jnp.minimum(v, partner),
                jnp.maximum(v, partner),
            )
            j //= 2
        k *= 2
    return v


def _med_body(nb, k_ref, g_ref, out_ref):
    i = pl.program_id(0)
    d = g_ref.shape[1]
    for j in range(nb):
        v = g_ref[j * MAXD:(j + 1) * MAXD, :]
        vs = _bitonic64(v)
        kj = k_ref[i * nb + j]
        eq = lax.broadcasted_iota(jnp.int32, (MAXD, d), 0) == kj
        out_ref[j, :] = jnp.sum(jnp.where(eq, vs, 0.0), axis=0)


def _median(g, karr):
    n = karr.shape[0]
    d = g.shape[1]
    nb = 8
    grid_spec = pltpu.PrefetchScalarGridSpec(
        num_scalar_prefetch=1,
        grid=(n // nb,),
        in_specs=[pl.BlockSpec((nb * MAXD, d), lambda i, k: (i, 0))],
        out_specs=pl.BlockSpec((nb, d), lambda i, k: (i, 0)),
    )
    return pl.pallas_call(
        functools.partial(_med_body, nb),
        grid_spec=grid_spec,
        out_shape=jax.ShapeDtypeStruct((n, d), jnp.float32),
    )(karr, g)


def kernel(feat, edge_index, weight, bias):
    n, _ = feat.shape
    do = weight.shape[1]
    e = edge_index.shape[1]
    src = edge_index[0]
    dst = edge_index[1]
    loop = jnp.arange(n, dtype=src.dtype)
    src_a = jnp.concatenate([src, loop])
    dst_a = jnp.concatenate([dst, loop])
    ep = e + n

    _, srcs = lax.sort_key_val(dst_a, src_a)
    counts = jnp.zeros((n,), jnp.int32).at[dst_a].add(1)
    offsets = jnp.cumsum(counts) - counts
    ceff = jnp.minimum(counts, MAXD)
    karr = ((ceff - 1) // 2).astype(jnp.int32)

    lane = jnp.arange(MAXD, dtype=jnp.int32)[None, :]
    pos = offsets[:, None] + lane
    valid = lane < counts[:, None]
    idx = jnp.where(valid, srcs[jnp.clip(pos, 0, ep - 1)], n).astype(jnp.int32)
    idx_flat = idx.reshape(-1)
    bp = 32 * 192 * -(-idx_flat.shape[0] // (32 * 192))
    idx_flat = jnp.concatenate(
        [idx_flat, jnp.full((bp - idx_flat.shape[0],), n, jnp.int32)]
    )

    h = _matmul(feat, weight, bias)
    h_ext = jnp.concatenate([h, jnp.full((10240 - n, do), jnp.inf, jnp.float32)])
    g = _sc_gather(h_ext, idx_flat)
    return _median(g, karr)


# speedup vs baseline: 9.9997x; 2.1720x over previous
import functools

import numpy as np
import jax
import jax.numpy as jnp
from jax import lax
from jax.experimental import pallas as pl
from jax.experimental.pallas import tpu as pltpu
from jax.experimental.pallas import tpu_sc as plsc

MAXD = 64


def _mm_body(f_ref, w_ref, b_ref, o_ref):
    o_ref[...] = (
        jnp.dot(f_ref[...], w_ref[...], preferred_element_type=jnp.float32)
        + b_ref[...]
    )


def _matmul(feat, weight, bias):
    n, di = feat.shape
    do = weight.shape[1]
    rb = 1000
    return pl.pallas_call(
        _mm_body,
        grid=(n // rb,),
        in_specs=[
            pl.BlockSpec((rb, di), lambda i: (i, 0)),
            pl.BlockSpec((di, do), lambda i: (0, 0)),
            pl.BlockSpec((1, do), lambda i: (0, 0)),
        ],
        out_specs=pl.BlockSpec((rb, do), lambda i: (i, 0)),
        out_shape=jax.ShapeDtypeStruct((n, do), jnp.float32),
    )(feat, weight, bias.reshape(1, do))


def _sc_gather(h_ext, idx_flat):
    info = plsc.get_sparse_core_info()
    nw = info.num_cores * info.num_subcores
    b = idx_flat.shape[0]
    d = h_ext.shape[1]
    b_per_w = b // nw
    k = 12
    ch = 16
    sup = k * ch
    nsup = b_per_w // sup
    mesh = plsc.VectorSubcoreMesh(core_axis_name="c", subcore_axis_name="s")

    nh = h_ext.shape[0]
    per_tile = nh // info.num_subcores

    @functools.partial(
        pl.kernel,
        out_type=jax.ShapeDtypeStruct((b, d), jnp.float32),
        mesh=mesh,
        scratch_types=[
            pltpu.VMEM((b_per_w,), jnp.int32),
            pltpu.VMEM((sup, d), jnp.float32),
            pltpu.VMEM_SHARED((nh, d), jnp.float32),
            pltpu.SemaphoreType.DMA,
        ],
    )
    def gk(h_hbm, idx_hbm, out_hbm, idx_v, buf, hs, gsem):
        sid = lax.axis_index("s")
        wid = sid * info.num_cores + lax.axis_index("c")
        base0 = wid * b_per_w

        tbase = sid * per_tile
        off = 0
        while off < per_tile:
            step = min(sup, per_tile - off)
            pltpu.sync_copy(h_hbm.at[pl.ds(tbase + off, step)],
                            buf.at[pl.ds(0, step)])
            pltpu.sync_copy(buf.at[pl.ds(0, step)],
                            hs.at[pl.ds(tbase + off, step)])
            off += step
        pltpu.sync_copy(idx_hbm.at[pl.ds(base0, b_per_w)], idx_v)
        plsc.subcore_barrier()

        def body(t, carry):
            base = t * sup
            for kk in range(k):
                pltpu.async_copy(
                    hs.at[idx_v.at[pl.ds(base + kk * ch, ch)]],
                    buf.at[pl.ds(kk * ch, ch)],
                    gsem,
                )
            for kk in range(k):
                pltpu.make_async_copy(
                    hs.at[idx_v.at[pl.ds(base + kk * ch, ch)]],
                    buf.at[pl.ds(kk * ch, ch)],
                    gsem,
                ).wait()
            pltpu.sync_copy(buf, out_hbm.at[pl.ds(base0 + base, sup)])
            return carry

        lax.fori_loop(0, nsup, body, 0)

    return gk(h_ext, idx_flat)


def _med_body(g_ref, m_ref, out_ref):
    v = [g_ref[s] for s in range(MAXD)]
    kk = 2
    while kk <= MAXD:
        j = kk // 2
        while j >= 1:
            for i in range(MAXD):
                l = i ^ j
                if l > i:
                    a, b = v[i], v[l]
                    mn = jnp.minimum(a, b)
                    mx = jnp.maximum(a, b)
                    if (i & kk) == 0:
                        v[i], v[l] = mn, mx
                    else:
                        v[i], v[l] = mx, mn
            j //= 2
        kk *= 2
    m = m_ref[0]
    acc = m[:, 0:1] * v[0]
    for s in range(1, MAXD):
        acc = acc + m[:, s:s + 1] * v[s]
    out_ref[...] = acc


def _median(g3, msel):
    nblk = msel.shape[0]
    nb = msel.shape[1]
    d = g3.shape[2]
    return pl.pallas_call(
        _med_body,
        grid=(nblk,),
        in_specs=[
            pl.BlockSpec((MAXD, nb, d), lambda i: (0, i, 0)),
            pl.BlockSpec((1, nb, MAXD), lambda i: (i, 0, 0)),
        ],
        out_specs=pl.BlockSpec((nb, d), lambda i: (i, 0)),
        out_shape=jax.ShapeDtypeStruct((nblk * nb, d), jnp.float32),
    )(g3, msel)


def kernel(feat, edge_index, weight, bias):
    n, _ = feat.shape
    do = weight.shape[1]
    e = edge_index.shape[1]
    src = edge_index[0]
    dst = edge_index[1]
    loop = jnp.arange(n, dtype=src.dtype)
    src_a = jnp.concatenate([src, loop])
    dst_a = jnp.concatenate([dst, loop])
    ep = e + n

    _, srcs = lax.sort_key_val(dst_a, src_a)
    counts = jnp.zeros((n,), jnp.int32).at[dst_a].add(1)
    offsets = jnp.cumsum(counts) - counts
    ceff = jnp.minimum(counts, MAXD)
    karr = ((ceff - 1) // 2).astype(jnp.int32)

    lane = jnp.arange(MAXD, dtype=jnp.int32)[None, :]
    pos = offsets[:, None] + lane
    valid = lane < counts[:, None]
    idx = jnp.where(valid, srcs[jnp.clip(pos, 0, ep - 1)], n).astype(jnp.int32)
    np_ = 96 * -(-n // 96)
    idx_t = jnp.pad(idx, ((0, np_ - n), (0, 0)), constant_values=n).T.reshape(-1)

    nb = 8
    msel = (
        karr.reshape(n // nb, nb)[:, :, None]
        == jnp.arange(MAXD, dtype=jnp.int32)[None, None, :]
    ).astype(jnp.float32)

    h = _matmul(feat, weight, bias)
    h_ext = jnp.concatenate(
        [h, jnp.full((10240 - n, do), 3.0e38, jnp.float32)]
    )
    g = _sc_gather(h_ext, idx_t)
    g3 = g.reshape(MAXD, np_, do)
    return _median(g3, msel)
